# Initial kernel scaffold; baseline (speedup 1.0000x reference)
import jax
import jax.numpy as jnp
from jax.experimental import pallas as pl


def kernel(x, edge_index, W, b):
    raise NotImplementedError("write your pallas kernel here")


# speedup vs baseline: 6.6045x; 6.6045x over previous
import functools

import jax
import jax.numpy as jnp
from jax import lax
from jax.experimental import pallas as pl
from jax.experimental.pallas import tpu as pltpu
from jax.experimental.pallas import tpu_sc as plsc

N = 10000
E = 320000
D = 128

NC = 2
NS = 16
NW = NC * NS
HD = D // NC
EPT = E // NW
CHUNK = 80
NCHUNK = EPT // CHUNK
AEPT = E // NS
ACHUNKS = AEPT // CHUNK
NPAD = 10240
NODES_PT = NPAD // NS

_mesh = plsc.VectorSubcoreMesh(core_axis_name="c", subcore_axis_name="s")


@functools.partial(
    pl.kernel,
    out_type=(
        jax.ShapeDtypeStruct((NC, NPAD), jnp.float32),
        jax.ShapeDtypeStruct((NC, NPAD), jnp.float32),
    ),
    mesh=_mesh,
    scratch_types=[
        pltpu.VMEM((NCHUNK, CHUNK), jnp.int32),
        pltpu.VMEM((NCHUNK, CHUNK), jnp.int32),
        pltpu.VMEM((CHUNK,), jnp.float32),
        pltpu.VMEM((NODES_PT,), jnp.float32),
        pltpu.VMEM_SHARED((NPAD,), jnp.float32),
        pltpu.VMEM_SHARED((NPAD,), jnp.float32),
    ],
)
def _deg_kernel(src_hbm, dst_hbm, outdeg_hbm, indeg_hbm,
                src_v, dst_v, ones_v, zeros_v, odeg_sh, ideg_sh):
    c = lax.axis_index("c")
    s = lax.axis_index("s")
    wid = c * NS + s
    for i in range(CHUNK // 16):
        ones_v[pl.ds(i * 16, 16)] = jnp.full((16,), 1.0, jnp.float32)
    def zfill(i, carry):
        zeros_v[pl.ds(i * 16, 16)] = jnp.zeros((16,), jnp.float32)
        return carry
    lax.fori_loop(0, NODES_PT // 16, zfill, 0)
    sl = pl.ds(s * NODES_PT, NODES_PT)
    pltpu.sync_copy(zeros_v, odeg_sh.at[sl])
    pltpu.sync_copy(zeros_v, ideg_sh.at[sl])
    pltpu.sync_copy(src_hbm.at[wid], src_v)
    pltpu.sync_copy(dst_hbm.at[wid], dst_v)
    plsc.subcore_barrier()
    def body(j, carry):
        pltpu.sync_copy(ones_v, odeg_sh.at[src_v.at[j]], add=True)
        pltpu.sync_copy(ones_v, ideg_sh.at[dst_v.at[j]], add=True)
        return carry
    lax.fori_loop(0, NCHUNK, body, 0)
    plsc.subcore_barrier()
    pltpu.sync_copy(odeg_sh.at[sl], outdeg_hbm.at[c, sl])
    pltpu.sync_copy(ideg_sh.at[sl], indeg_hbm.at[c, sl])


def _scale_body(x_ref, po_ref, h_ref):
    deg = po_ref[0, :N] + po_ref[1, :N]
    norm = lax.rsqrt(jnp.maximum(deg, 1.0))
    h = x_ref[...] * norm[:, None]
    h_ref[0, :, :] = h[:, :HD]
    h_ref[1, :, :] = h[:, HD:]


_scale = pl.pallas_call(
    _scale_body,
    out_shape=jax.ShapeDtypeStruct((NC, N, HD), jnp.float32),
)


@functools.partial(
    pl.kernel,
    out_type=jax.ShapeDtypeStruct((NC, NPAD, HD), jnp.float32),
    mesh=_mesh,
    scratch_types=[
        pltpu.VMEM((ACHUNKS, CHUNK), jnp.int32),
        pltpu.VMEM((ACHUNKS, CHUNK), jnp.int32),
        pltpu.VMEM((CHUNK, HD), jnp.float32),
        pltpu.VMEM((64, HD), jnp.float32),
        pltpu.VMEM_SHARED((NPAD, HD), jnp.float32),
        pltpu.SemaphoreType.DMA,
    ],
    compiler_params=pltpu.CompilerParams(use_tc_tiling_on_sc=False),
)
def _agg_kernel(h_hbm, src_hbm, dst_hbm, out_hbm,
                src_v, dst_v, rows_v, zeros_v, agg_sh, sem):
    c = lax.axis_index("c")
    s = lax.axis_index("s")
    def zfill(r, carry):
        for k in range(HD // 16):
            zeros_v[r, pl.ds(k * 16, 16)] = jnp.zeros((16,), jnp.float32)
        return carry
    lax.fori_loop(0, 64, zfill, 0)
    for i in range(NODES_PT // 64):
        pltpu.sync_copy(zeros_v, agg_sh.at[pl.ds(s * NODES_PT + i * 64, 64)])
    pltpu.sync_copy(src_hbm.at[s], src_v)
    pltpu.sync_copy(dst_hbm.at[s], dst_v)
    plsc.subcore_barrier()
    hc = h_hbm.at[c]
    def body(j, carry):
        pltpu.async_copy(hc.at[src_v.at[j]], rows_v, sem).wait()
        pltpu.sync_copy(rows_v, agg_sh.at[dst_v.at[j]], add=True)
        return carry
    lax.fori_loop(0, ACHUNKS, body, 0)
    plsc.subcore_barrier()
    sl = pl.ds(s * NODES_PT, NODES_PT)
    pltpu.sync_copy(agg_sh.at[sl], out_hbm.at[c, sl])


def _final_body(p_ref, w_ref, b_ref, pi_ref, o_ref):
    agg = jnp.concatenate([p_ref[0, :N, :], p_ref[1, :N, :]], axis=1)
    deg = pi_ref[0, :N] + pi_ref[1, :N]
    norm = lax.rsqrt(jnp.maximum(deg, 1.0))
    rst = jnp.dot(agg, w_ref[...], preferred_element_type=jnp.float32)
    o_ref[...] = rst * norm[:, None] + b_ref[...][None, :]


_final = pl.pallas_call(
    _final_body,
    out_shape=jax.ShapeDtypeStruct((N, D), jnp.float32),
)


def kernel(x, edge_index, W, b):
    src = edge_index[0].astype(jnp.int32)
    dst = edge_index[1].astype(jnp.int32)
    src_d = src.reshape(NW, NCHUNK, CHUNK)
    dst_d = dst.reshape(NW, NCHUNK, CHUNK)
    src_a = src.reshape(NS, ACHUNKS, CHUNK)
    dst_a = dst.reshape(NS, ACHUNKS, CHUNK)
    outdeg_p, indeg_p = _deg_kernel(src_d, dst_d)
    h = _scale(x, outdeg_p)
    parts = _agg_kernel(h, src_a, dst_a)
    return _final(parts, W, b, indeg_p)


# speedup vs baseline: 10.0510x; 1.5218x over previous
import functools

import jax
import jax.numpy as jnp
from jax import lax
from jax.experimental import pallas as pl
from jax.experimental.pallas import tpu as pltpu
from jax.experimental.pallas import tpu_sc as plsc

N = 10000
E = 320000
D = 128

NC = 2
NS = 16
NW = NC * NS
HD = D // NC
EPT = E // NW
CHUNK = 80
NCHUNK = EPT // CHUNK
AEPT = E // NS
ACHUNKS = AEPT // CHUNK
NPAD = 10240
NODES_PT = NPAD // NS

_mesh = plsc.VectorSubcoreMesh(core_axis_name="c", subcore_axis_name="s")


@functools.partial(
    pl.kernel,
    out_type=(
        jax.ShapeDtypeStruct((NC, NPAD), jnp.float32),
        jax.ShapeDtypeStruct((NC, NPAD), jnp.float32),
    ),
    mesh=_mesh,
    scratch_types=[
        pltpu.VMEM((NCHUNK, CHUNK), jnp.int32),
        pltpu.VMEM((NCHUNK, CHUNK), jnp.int32),
        pltpu.VMEM((CHUNK,), jnp.float32),
        pltpu.VMEM((NODES_PT,), jnp.float32),
        pltpu.VMEM_SHARED((NPAD,), jnp.float32),
        pltpu.VMEM_SHARED((NPAD,), jnp.float32),
    ],
)
def _deg_kernel(src_hbm, dst_hbm, outdeg_hbm, indeg_hbm,
                src_v, dst_v, ones_v, zeros_v, odeg_sh, ideg_sh):
    c = lax.axis_index("c")
    s = lax.axis_index("s")
    wid = c * NS + s
    for i in range(CHUNK // 16):
        ones_v[pl.ds(i * 16, 16)] = jnp.full((16,), 1.0, jnp.float32)
    def zfill(i, carry):
        zeros_v[pl.ds(i * 16, 16)] = jnp.zeros((16,), jnp.float32)
        return carry
    lax.fori_loop(0, NODES_PT // 16, zfill, 0)
    sl = pl.ds(s * NODES_PT, NODES_PT)
    pltpu.sync_copy(zeros_v, odeg_sh.at[sl])
    pltpu.sync_copy(zeros_v, ideg_sh.at[sl])
    pltpu.sync_copy(src_hbm.at[wid], src_v)
    pltpu.sync_copy(dst_hbm.at[wid], dst_v)
    plsc.subcore_barrier()
    def body(j, carry):
        pltpu.sync_copy(ones_v, odeg_sh.at[src_v.at[j]], add=True)
        pltpu.sync_copy(ones_v, ideg_sh.at[dst_v.at[j]], add=True)
        return carry
    lax.fori_loop(0, NCHUNK, body, 0)
    plsc.subcore_barrier()
    pltpu.sync_copy(odeg_sh.at[sl], outdeg_hbm.at[c, sl])
    pltpu.sync_copy(ideg_sh.at[sl], indeg_hbm.at[c, sl])


def _scale_body(x_ref, po_ref, h_ref):
    deg = po_ref[0, :N] + po_ref[1, :N]
    norm = lax.rsqrt(jnp.maximum(deg, 1.0))
    h = x_ref[...] * norm[:, None]
    h_ref[0, :, :] = h[:, :HD]
    h_ref[1, :, :] = h[:, HD:]


_scale = pl.pallas_call(
    _scale_body,
    out_shape=jax.ShapeDtypeStruct((NC, N, HD), jnp.float32),
)


@functools.partial(
    pl.kernel,
    out_type=jax.ShapeDtypeStruct((NC, NPAD, HD), jnp.float32),
    mesh=_mesh,
    scratch_types=[
        pltpu.VMEM((ACHUNKS, CHUNK), jnp.int32),
        pltpu.VMEM((ACHUNKS, CHUNK), jnp.int32),
        pltpu.VMEM((CHUNK, HD), jnp.float32),
        pltpu.VMEM((CHUNK, HD), jnp.float32),
        pltpu.VMEM((64, HD), jnp.float32),
        pltpu.VMEM_SHARED((NPAD, HD), jnp.float32),
        pltpu.SemaphoreType.DMA,
        pltpu.SemaphoreType.DMA,
    ],
    compiler_params=pltpu.CompilerParams(use_tc_tiling_on_sc=False),
)
def _agg_kernel(h_hbm, src_hbm, dst_hbm, out_hbm,
                src_v, dst_v, rows0_v, rows1_v, zeros_v, agg_sh, sem0, sem1):
    c = lax.axis_index("c")
    s = lax.axis_index("s")
    def zfill(r, carry):
        for k in range(HD // 16):
            zeros_v[r, pl.ds(k * 16, 16)] = jnp.zeros((16,), jnp.float32)
        return carry
    lax.fori_loop(0, 64, zfill, 0)
    for i in range(NODES_PT // 64):
        pltpu.sync_copy(zeros_v, agg_sh.at[pl.ds(s * NODES_PT + i * 64, 64)])
    pltpu.sync_copy(src_hbm.at[s], src_v)
    pltpu.sync_copy(dst_hbm.at[s], dst_v)
    plsc.subcore_barrier()
    hc = h_hbm.at[c]

    def start_g(jj, buf, sem):
        pltpu.async_copy(hc.at[src_v.at[jj]], buf, sem)

    def wait_g(buf, sem):
        pltpu.make_async_copy(hc.at[src_v.at[0]], buf, sem).wait()

    PAIRS = ACHUNKS // 2
    start_g(0, rows0_v, sem0)

    def body(j, carry):
        start_g(2 * j + 1, rows1_v, sem1)
        wait_g(rows0_v, sem0)
        pltpu.sync_copy(rows0_v, agg_sh.at[dst_v.at[2 * j]], add=True)
        pl.when(j < PAIRS - 1)(lambda: start_g(2 * j + 2, rows0_v, sem0))
        wait_g(rows1_v, sem1)
        pltpu.sync_copy(rows1_v, agg_sh.at[dst_v.at[2 * j + 1]], add=True)
        return carry
    lax.fori_loop(0, PAIRS, body, 0)
    plsc.subcore_barrier()
    sl = pl.ds(s * NODES_PT, NODES_PT)
    pltpu.sync_copy(agg_sh.at[sl], out_hbm.at[c, sl])


def _final_body(p_ref, w_ref, b_ref, pi_ref, o_ref):
    agg = jnp.concatenate([p_ref[0, :N, :], p_ref[1, :N, :]], axis=1)
    deg = pi_ref[0, :N] + pi_ref[1, :N]
    norm = lax.rsqrt(jnp.maximum(deg, 1.0))
    rst = jnp.dot(agg, w_ref[...], preferred_element_type=jnp.float32)
    o_ref[...] = rst * norm[:, None] + b_ref[...][None, :]


_final = pl.pallas_call(
    _final_body,
    out_shape=jax.ShapeDtypeStruct((N, D), jnp.float32),
)


def kernel(x, edge_index, W, b):
    src = edge_index[0].astype(jnp.int32)
    dst = edge_index[1].astype(jnp.int32)
    src_d = src.reshape(NW, NCHUNK, CHUNK)
    dst_d = dst.reshape(NW, NCHUNK, CHUNK)
    src_a = src.reshape(NS, ACHUNKS, CHUNK)
    dst_a = dst.reshape(NS, ACHUNKS, CHUNK)
    outdeg_p, indeg_p = _deg_kernel(src_d, dst_d)
    h = _scale(x, outdeg_p)
    parts = _agg_kernel(h, src_a, dst_a)
    return _final(parts, W, b, indeg_p)
